# Initial kernel scaffold; baseline (speedup 1.0000x reference)
import jax
import jax.numpy as jnp
from jax.experimental import pallas as pl


def kernel(inputs, img_info, w_cls, b_cls, w_reg, b_reg):
    raise NotImplementedError("write your pallas kernel here")


# speedup vs baseline: 11.1036x; 11.1036x over previous
import jax
import jax.numpy as jnp
import numpy as np
from jax.experimental import pallas as pl
from jax.experimental.pallas import tpu as pltpu

_STRIDE = 4
_ANCHOR_SCALE = 8.0
_ASPECTS = [(1.0, 1.0), (1.4, 0.7), (0.7, 1.4)]
_PRE = 2000
_POST = 1000
_THRESH = 0.7
_CLIP = float(np.log(1000.0 / 16.0))
_NPAD = 2048


def _anchors_np(feat_h, feat_w):
    base = _ANCHOR_SCALE * _STRIDE
    ys = (np.arange(feat_h, dtype=np.float32) + 0.5) * _STRIDE
    xs = (np.arange(feat_w, dtype=np.float32) + 0.5) * _STRIDE
    cy, cx = np.meshgrid(ys, xs, indexing='ij')
    per = []
    for ax, ay in _ASPECTS:
        hh = base * ay / 2.0
        hw = base * ax / 2.0
        per.append(np.stack([cy - hh, cx - hw, cy + hh, cx + hw], axis=-1))
    return np.stack(per, axis=2).reshape(-1, 4).astype(np.float32)


def _head_kernel(x_ref, w_ref, b_ref, o_ref):
    o_ref[...] = jnp.dot(x_ref[...], w_ref[...],
                         preferred_element_type=jnp.float32) + b_ref[...]


def _run_head(x2d, wcat, bcat):
    m = x2d.shape[0]
    bm = 4368
    grid = m // bm
    return pl.pallas_call(
        _head_kernel,
        grid=(grid,),
        in_specs=[
            pl.BlockSpec((bm, 256), lambda i: (i, 0)),
            pl.BlockSpec((256, 15), lambda i: (0, 0)),
            pl.BlockSpec((1, 15), lambda i: (0, 0)),
        ],
        out_specs=pl.BlockSpec((bm, 15), lambda i: (i, 0)),
        out_shape=jax.ShapeDtypeStruct((m, 15), jnp.float32),
    )(x2d, wcat, bcat)


def _nms_kernel(a_ref, d_ref, p_ref, hw_ref, rois_ref, sc_ref, s_ref):
    rois_ref[...] = jnp.zeros_like(rois_ref)
    sc_ref[...] = jnp.zeros_like(sc_ref)

    ha = a_ref[2:3, :] - a_ref[0:1, :]
    wa = a_ref[3:4, :] - a_ref[1:2, :]
    cya = a_ref[0:1, :] + 0.5 * ha
    cxa = a_ref[1:2, :] + 0.5 * wa
    dh = jnp.minimum(d_ref[2:3, :], _CLIP)
    dw = jnp.minimum(d_ref[3:4, :], _CLIP)
    cy = d_ref[0:1, :] * ha + cya
    cx = d_ref[1:2, :] * wa + cxa
    bh = jnp.exp(dh) * ha
    bw = jnp.exp(dw) * wa
    hmax = hw_ref[0, 0] - 1.0
    wmax = hw_ref[0, 1] - 1.0
    y1 = jnp.clip(cy - 0.5 * bh, 0.0, hmax)
    x1 = jnp.clip(cx - 0.5 * bw, 0.0, wmax)
    y2 = jnp.clip(cy + 0.5 * bh, 0.0, hmax)
    x2 = jnp.clip(cx + 0.5 * bw, 0.0, wmax)
    s_ref[0:1, :] = y1
    s_ref[1:2, :] = x1
    s_ref[2:3, :] = y2
    s_ref[3:4, :] = x2
    s_ref[4:5, :] = (y2 - y1) * (x2 - x1)
    lane0 = jax.lax.broadcasted_iota(jnp.int32, (1, _NPAD), 1)
    s_ref[5:6, :] = jnp.where(lane0 < _PRE, 1.0, 0.0)

    def body(i, cnt):
        lane = jax.lax.broadcasted_iota(jnp.int32, (1, _NPAD), 1)
        onehot = (lane == i).astype(jnp.float32)
        keep = s_ref[5:6, :]
        ki = jnp.sum(keep * onehot)
        yy1 = s_ref[0:1, :]
        xx1 = s_ref[1:2, :]
        yy2 = s_ref[2:3, :]
        xx2 = s_ref[3:4, :]
        bi_y1 = jnp.sum(yy1 * onehot)
        bi_x1 = jnp.sum(xx1 * onehot)
        bi_y2 = jnp.sum(yy2 * onehot)
        bi_x2 = jnp.sum(xx2 * onehot)
        ai = (bi_y2 - bi_y1) * (bi_x2 - bi_x1)
        inter_h = jnp.maximum(jnp.minimum(yy2, bi_y2) - jnp.maximum(yy1, bi_y1), 0.0)
        inter_w = jnp.maximum(jnp.minimum(xx2, bi_x2) - jnp.maximum(xx1, bi_x1), 0.0)
        inter = inter_h * inter_w
        union = s_ref[4:5, :] + ai - inter
        iou = inter / jnp.maximum(union, 1e-8)
        alive = ki > 0.5
        sup = (iou > _THRESH) & (lane > i) & alive
        s_ref[5:6, :] = jnp.where(sup, 0.0, keep)

        emit = alive & (cnt < _POST)

        @pl.when(emit)
        def _():
            row = jnp.concatenate([
                bi_y1.reshape(1, 1), bi_x1.reshape(1, 1),
                bi_y2.reshape(1, 1), bi_x2.reshape(1, 1)], axis=1)
            rois_ref[pl.ds(cnt, 1), :] = row
            si = jnp.sum(p_ref[0:1, :] * onehot)
            sc_ref[pl.ds(cnt, 1), :] = si.reshape(1, 1)

        return cnt + jnp.where(emit, 1, 0).astype(jnp.int32)

    jax.lax.fori_loop(0, _PRE, body, jnp.int32(0))


def _run_nms(anc_row, del_row, p_row, hw):
    return pl.pallas_call(
        _nms_kernel,
        out_shape=[jax.ShapeDtypeStruct((1024, 4), jnp.float32),
                   jax.ShapeDtypeStruct((1024, 1), jnp.float32)],
        scratch_shapes=[pltpu.VMEM((8, _NPAD), jnp.float32)],
    )(anc_row, del_row, p_row, hw)


def kernel(inputs, img_info, w_cls, b_cls, w_reg, b_reg):
    b, h, w, c = inputs.shape
    x2d = inputs.reshape(b * h * w, c)
    wcat = jnp.concatenate([w_cls, w_reg], axis=1)
    bcat = jnp.concatenate([b_cls, b_reg]).reshape(1, 15)
    out15 = _run_head(x2d, wcat, bcat)
    cls_scores = out15[:, :3].reshape(b, h, w, 3)
    bbox_preds = out15[:, 3:15].reshape(b, h, w, 12)

    probs = jax.nn.sigmoid(out15[:, :3].reshape(-1))
    top_p, top_idx = jax.lax.top_k(probs, _PRE)
    anchors = jnp.asarray(_anchors_np(h, w))
    anc_top = anchors[top_idx]
    del_top = out15[:, 3:15].reshape(-1, 4)[top_idx]

    anc_row = jnp.zeros((4, _NPAD), jnp.float32).at[:, :_PRE].set(anc_top.T)
    del_row = jnp.zeros((4, _NPAD), jnp.float32).at[:, :_PRE].set(del_top.T)
    p_row = jnp.zeros((1, _NPAD), jnp.float32).at[0, :_PRE].set(top_p)
    hw = img_info[:1, :2]

    rois_p, sc_p = _run_nms(anc_row, del_row, p_row, hw)
    rois = rois_p[:_POST].reshape(1, _POST, 4)
    roi_scores = sc_p[:_POST, 0].reshape(1, _POST)
    return cls_scores, bbox_preds, rois, roi_scores


# speedup vs baseline: 13.7556x; 1.2388x over previous
import jax
import jax.numpy as jnp
import numpy as np
from jax.experimental import pallas as pl
from jax.experimental.pallas import tpu as pltpu

_STRIDE = 4
_ANCHOR_SCALE = 8.0
_ASPECTS = [(1.0, 1.0), (1.4, 0.7), (0.7, 1.4)]
_PRE = 2000
_POST = 1000
_THRESH = 0.7
_CLIP = float(np.log(1000.0 / 16.0))
_NPAD = 2048


def _anchors_np(feat_h, feat_w):
    base = _ANCHOR_SCALE * _STRIDE
    ys = (np.arange(feat_h, dtype=np.float32) + 0.5) * _STRIDE
    xs = (np.arange(feat_w, dtype=np.float32) + 0.5) * _STRIDE
    cy, cx = np.meshgrid(ys, xs, indexing='ij')
    per = []
    for ax, ay in _ASPECTS:
        hh = base * ay / 2.0
        hw = base * ax / 2.0
        per.append(np.stack([cy - hh, cx - hw, cy + hh, cx + hw], axis=-1))
    return np.stack(per, axis=2).reshape(-1, 4).astype(np.float32)


def _head_kernel(x_ref, w_ref, b_ref, o_ref):
    o_ref[...] = jnp.dot(x_ref[...], w_ref[...],
                         preferred_element_type=jnp.float32) + b_ref[...]


def _run_head(x2d, wcat, bcat):
    m = x2d.shape[0]
    bm = 4368
    grid = m // bm
    return pl.pallas_call(
        _head_kernel,
        grid=(grid,),
        in_specs=[
            pl.BlockSpec((bm, 256), lambda i: (i, 0)),
            pl.BlockSpec((256, 15), lambda i: (0, 0)),
            pl.BlockSpec((1, 15), lambda i: (0, 0)),
        ],
        out_specs=pl.BlockSpec((bm, 15), lambda i: (i, 0)),
        out_shape=jax.ShapeDtypeStruct((m, 15), jnp.float32),
    )(x2d, wcat, bcat)


def _decode_kernel(a_ref, d_ref, hw_ref, o_ref):
    ha = a_ref[2:3, :] - a_ref[0:1, :]
    wa = a_ref[3:4, :] - a_ref[1:2, :]
    cya = a_ref[0:1, :] + 0.5 * ha
    cxa = a_ref[1:2, :] + 0.5 * wa
    dh = jnp.minimum(d_ref[2:3, :], _CLIP)
    dw = jnp.minimum(d_ref[3:4, :], _CLIP)
    cy = d_ref[0:1, :] * ha + cya
    cx = d_ref[1:2, :] * wa + cxa
    bh = jnp.exp(dh) * ha
    bw = jnp.exp(dw) * wa
    hmax = hw_ref[0, 0] - 1.0
    wmax = hw_ref[0, 1] - 1.0
    y1 = jnp.clip(cy - 0.5 * bh, 0.0, hmax)
    x1 = jnp.clip(cx - 0.5 * bw, 0.0, wmax)
    y2 = jnp.clip(cy + 0.5 * bh, 0.0, hmax)
    x2 = jnp.clip(cx + 0.5 * bw, 0.0, wmax)
    o_ref[0:1, :] = y1
    o_ref[1:2, :] = x1
    o_ref[2:3, :] = y2
    o_ref[3:4, :] = x2
    o_ref[4:5, :] = (y2 - y1) * (x2 - x1)
    o_ref[5:8, :] = jnp.zeros((3, _NPAD), jnp.float32)


def _nms_kernel(br_ref, bc_ref, pc_ref, rois_ref, sc_ref, keep_ref):
    rois_ref[...] = jnp.zeros_like(rois_ref)
    sc_ref[...] = jnp.zeros_like(sc_ref)
    lane0 = jax.lax.broadcasted_iota(jnp.int32, (1, _NPAD), 1)
    keep_ref[...] = jnp.where(lane0 < _PRE, 1.0, 0.0)

    def cond(carry):
        i, cnt = carry
        return (i < _PRE) & (cnt < _POST)

    def body(carry):
        i, cnt = carry
        lane = jax.lax.broadcasted_iota(jnp.int32, (1, _NPAD), 1)
        onehot = (lane == i).astype(jnp.float32)
        ki = jnp.sum(keep_ref[...] * onehot)
        alive = ki > 0.5

        @pl.when(alive)
        def _():
            bi = bc_ref[pl.ds(i, 1), :]
            bi_y1 = bi[0, 0]
            bi_x1 = bi[0, 1]
            bi_y2 = bi[0, 2]
            bi_x2 = bi[0, 3]
            ai = (bi_y2 - bi_y1) * (bi_x2 - bi_x1)
            yy1 = br_ref[0:1, :]
            xx1 = br_ref[1:2, :]
            yy2 = br_ref[2:3, :]
            xx2 = br_ref[3:4, :]
            ih = jnp.maximum(jnp.minimum(yy2, bi_y2) - jnp.maximum(yy1, bi_y1), 0.0)
            iw = jnp.maximum(jnp.minimum(xx2, bi_x2) - jnp.maximum(xx1, bi_x1), 0.0)
            inter = ih * iw
            union = br_ref[4:5, :] + ai - inter
            iou = inter / jnp.maximum(union, 1e-8)
            sup = (iou > _THRESH) & (lane > i)
            keep_ref[...] = jnp.where(sup, 0.0, keep_ref[...])
            rois_ref[pl.ds(cnt, 1), :] = bi
            sc_ref[pl.ds(cnt, 1), :] = pc_ref[pl.ds(i, 1), :]

        return i + 1, cnt + jnp.where(alive, 1, 0).astype(jnp.int32)

    jax.lax.while_loop(cond, body, (jnp.int32(0), jnp.int32(0)))


def _run_nms(anc_row, del_row, p_col, hw):
    boxes_row = pl.pallas_call(
        _decode_kernel,
        out_shape=jax.ShapeDtypeStruct((8, _NPAD), jnp.float32),
    )(anc_row, del_row, hw)
    boxes_col = boxes_row[0:4].T
    return pl.pallas_call(
        _nms_kernel,
        out_shape=[jax.ShapeDtypeStruct((1024, 4), jnp.float32),
                   jax.ShapeDtypeStruct((1024, 1), jnp.float32)],
        scratch_shapes=[pltpu.VMEM((1, _NPAD), jnp.float32)],
    )(boxes_row, boxes_col, p_col)


def kernel(inputs, img_info, w_cls, b_cls, w_reg, b_reg):
    b, h, w, c = inputs.shape
    x2d = inputs.reshape(b * h * w, c)
    wcat = jnp.concatenate([w_cls, w_reg], axis=1)
    bcat = jnp.concatenate([b_cls, b_reg]).reshape(1, 15)
    out15 = _run_head(x2d, wcat, bcat)
    cls_scores = out15[:, :3].reshape(b, h, w, 3)
    bbox_preds = out15[:, 3:15].reshape(b, h, w, 12)

    probs = jax.nn.sigmoid(out15[:, :3].reshape(-1))
    top_p, top_idx = jax.lax.top_k(probs, _PRE)
    anchors = jnp.asarray(_anchors_np(h, w))
    anc_top = anchors[top_idx]
    del_top = out15[:, 3:15].reshape(-1, 4)[top_idx]

    anc_row = jnp.zeros((4, _NPAD), jnp.float32).at[:, :_PRE].set(anc_top.T)
    del_row = jnp.zeros((4, _NPAD), jnp.float32).at[:, :_PRE].set(del_top.T)
    p_col = jnp.zeros((_NPAD, 1), jnp.float32).at[:_PRE, 0].set(top_p)
    hw = img_info[:1, :2]

    rois_p, sc_p = _run_nms(anc_row, del_row, p_col, hw)
    rois = rois_p[:_POST].reshape(1, _POST, 4)
    roi_scores = sc_p[:_POST, 0].reshape(1, _POST)
    return cls_scores, bbox_preds, rois, roi_scores
